# Initial kernel scaffold; baseline (speedup 1.0000x reference)
import jax
import jax.numpy as jnp
from jax.experimental import pallas as pl


def kernel(length_q, length_k, pe):
    raise NotImplementedError("write your pallas kernel here")


# speedup vs baseline: 14.3184x; 14.3184x over previous
import jax
import jax.numpy as jnp
from jax.experimental import pallas as pl
from jax.experimental.pallas import tpu as pltpu

_LQ = 1024
_LK = 1024
_D = 256
_ROWS = 9
_MAXK = 4
_T = 4096
_MID = _T // 2 - 2
_BQ = 4


def _body(off_ref, pe_ref, out_ref, t8_ref):
    i = pl.program_id(0)

    @pl.when(i == 0)
    def _build():
        u = jax.lax.broadcasted_iota(jnp.int32, (_T, _D), 0)
        for s in range(8):
            c = jnp.clip(u + (s - _MID), -_MAXK, _MAXK) + _MAXK
            acc = jnp.zeros((_T, _D), jnp.float32)
            for r in range(_ROWS):
                acc = jnp.where(c == r, pe_ref[r, :][None, :], acc)
            t8_ref[s] = acc

    off = off_ref[0]
    for j in range(_BQ):
        q = i * _BQ + j
        start = jnp.clip(_MID - q + off, 0, _T - _LK)
        s = jax.lax.rem(start, 8)
        base = pl.multiple_of(start - s, 8)
        out_ref[j] = t8_ref[s, pl.ds(base, _LK), :]


def kernel(length_q, length_k, pe):
    off = jnp.asarray(length_k - length_q, jnp.int32).reshape((1,))
    return pl.pallas_call(
        _body,
        grid=(_LQ // _BQ,),
        in_specs=[
            pl.BlockSpec(memory_space=pltpu.SMEM),
            pl.BlockSpec((_ROWS, _D), lambda i: (0, 0)),
        ],
        out_specs=pl.BlockSpec((_BQ, _LK, _D), lambda i: (i, 0, 0)),
        out_shape=jax.ShapeDtypeStruct((_LQ, _LK, _D), jnp.float32),
        scratch_shapes=[pltpu.VMEM((8, _T, _D), jnp.float32)],
    )(off, pe)
